# baseline (device time: 6037 ns/iter reference)
import jax
import jax.numpy as jnp
from jax import lax
from jax.experimental import pallas as pl
from jax.experimental.pallas import tpu as pltpu


def kernel(x):
    m, n = x.shape
    nh = n // 2

    def body(x_ref, out_ref, send_ref, recv_ref, send_sems, recv_sems):
        my_x = lax.axis_index("x")
        my_y = lax.axis_index("y")
        peer = (1 - my_x, my_y)

        barrier_sem = pltpu.get_barrier_semaphore()
        pl.semaphore_signal(
            barrier_sem, inc=1, device_id=peer,
            device_id_type=pl.DeviceIdType.MESH,
        )

        def exchange(slot):
            return pltpu.make_async_remote_copy(
                src_ref=send_ref.at[slot],
                dst_ref=recv_ref.at[slot],
                send_sem=send_sems.at[slot],
                recv_sem=recv_sems.at[slot],
                device_id=peer,
                device_id_type=pl.DeviceIdType.MESH,
            )

        send_ref[0, :, :] = jnp.max(
            x_ref[:, pl.ds(0, nh)], axis=0, keepdims=True
        )
        pl.semaphore_wait(barrier_sem, 1)
        rdma_a = exchange(0)
        rdma_a.start()

        send_ref[1, :, :] = jnp.max(
            x_ref[:, pl.ds(nh, nh)], axis=0, keepdims=True
        )
        rdma_b = exchange(1)
        rdma_b.start()

        rdma_a.wait()
        out_ref[:, pl.ds(0, nh)] = jnp.maximum(
            send_ref[0, :, :], recv_ref[0, :, :]
        )
        rdma_b.wait()
        out_ref[:, pl.ds(nh, nh)] = jnp.maximum(
            send_ref[1, :, :], recv_ref[1, :, :]
        )

    return pl.pallas_call(
        body,
        out_shape=jax.ShapeDtypeStruct((1, n), x.dtype),
        in_specs=[pl.BlockSpec(memory_space=pltpu.VMEM)],
        out_specs=pl.BlockSpec(memory_space=pltpu.VMEM),
        scratch_shapes=[
            pltpu.VMEM((2, 1, nh), x.dtype),
            pltpu.VMEM((2, 1, nh), x.dtype),
            pltpu.SemaphoreType.DMA((2,)),
            pltpu.SemaphoreType.DMA((2,)),
        ],
        compiler_params=pltpu.CompilerParams(collective_id=0),
    )(x)
